# baseline (speedup 1.0000x reference)
import functools

import jax
import jax.numpy as jnp
from jax import lax
from jax.experimental import pallas as pl
from jax.experimental.pallas import tpu as pltpu
from jax.experimental.pallas import tpu_sc as plsc

_PLACEHOLDER = -1
_NEG_INF = float("-inf")


def _sc_gather(ids_flat, lflat, dflat, V, B):
    R = ids_flat.shape[0]
    info = plsc.get_sparse_core_info()
    nw = info.num_cores * info.num_subcores
    per = R // nw
    mesh = plsc.VectorSubcoreMesh(core_axis_name="c", subcore_axis_name="s")

    @functools.partial(
        pl.kernel,
        mesh=mesh,
        out_type=(
            jax.ShapeDtypeStruct((R,), jnp.float32),
            jax.ShapeDtypeStruct((R,), jnp.float32),
        ),
        scratch_types=[
            pltpu.VMEM((per,), jnp.int32),
            pltpu.VMEM((per,), jnp.int32),
            pltpu.VMEM((per,), jnp.float32),
            pltpu.VMEM((per,), jnp.float32),
            pltpu.SemaphoreType.DMA,
            pltpu.SemaphoreType.DMA,
        ],
    )
    def k(ids_hbm, lf_hbm, df_hbm, lat_hbm, dat_hbm,
          idx_v, flat_v, lat_v, dat_v, sem1, sem2):
        wid = lax.axis_index("s") * info.num_cores + lax.axis_index("c")
        base = wid * per
        pltpu.sync_copy(ids_hbm.at[pl.ds(base, per)], idx_v)
        w_per_s = B // per
        s_w = wid // w_per_s
        b0 = (wid - s_w * w_per_s) * per
        flat_v[...] = ((s_w * V + idx_v[...]) * B + b0
                       + lax.iota(jnp.int32, per))
        cp1 = pltpu.async_copy(lf_hbm.at[flat_v], lat_v, sem1)
        cp2 = pltpu.async_copy(df_hbm.at[flat_v], dat_v, sem2)
        cp1.wait()
        cp2.wait()
        pltpu.sync_copy(lat_v, lat_hbm.at[pl.ds(base, per)])
        pltpu.sync_copy(dat_v, dat_hbm.at[pl.ds(base, per)])

    return k(ids_flat, lflat, dflat)


def _k1_stats_scan(tl_t, ids_t, lat_t, dat_t, u_t, bonus_r, greedy_r, C):
    S, V, B = tl_t.shape
    NJ = V // C

    def body(lref, iref, latref, datref, uref, bref, gref,
             outref, nrref, laref, ssref, wrref, msref, zsref,
             mall, zall, aall, macc, zacc, aacc):
        s_id = pl.program_id(0)
        j = pl.program_id(1)
        X = lref[0]
        lm = jnp.max(X, axis=0, keepdims=True)
        iov = lax.broadcasted_iota(jnp.int32, (C, B), 0) + j * C
        lam = jnp.min(jnp.where(X == lm, iov, V), axis=0, keepdims=True)

        @pl.when(j == 0)
        def _():
            macc[...] = lm
            zacc[...] = jnp.sum(jnp.exp(X - lm), axis=0, keepdims=True)
            aacc[...] = lam

        @pl.when(j > 0)
        def _():
            mo = macc[...]
            mn = jnp.maximum(mo, lm)
            zacc[...] = (zacc[...] * jnp.exp(mo - mn)
                         + jnp.sum(jnp.exp(X - mn), axis=0, keepdims=True))
            aacc[...] = jnp.where(lm > mo, lam, aacc[...])
            macc[...] = mn

        @pl.when(j == NJ - 1)
        def _():
            mall[pl.ds(s_id, 1), :] = macc[...]
            zall[pl.ds(s_id, 1), :] = zacc[...]
            aall[pl.ds(s_id, 1), :] = aacc[...]

        @pl.when((j == NJ - 1) & (s_id == S - 1))
        def _():
            m = mall[...]
            z = zall[...]
            am = aall[...]
            ids = iref[...]
            lat = latref[...]
            dat = datref[...]
            u = uref[...]
            bonus = bref[...]
            greedy = gref[...] != 0

            t = jnp.exp(lat - m) / z
            acc = (dat > 0.0) & (
                (t / jnp.where(dat > 0.0, dat, 1.0)) >= u)
            match = ids == am

            ones = jnp.ones((1, B), dtype=jnp.bool_)
            prev_g = ones
            prev_r = ones
            numacc_g = jnp.zeros((1, B), dtype=jnp.int32)
            numacc_r = jnp.zeros((1, B), dtype=jnp.int32)
            neg1 = jnp.full((1, B), _PLACEHOLDER, dtype=jnp.int32)
            for s in range(S):
                acc_s = acc[s:s + 1, :]
                match_s = match[s:s + 1, :]
                am_s = am[s:s + 1, :]
                ids_s = ids[s:s + 1, :]
                tok_g = jnp.where(prev_g, am_s, neg1)
                tok_r = jnp.where(prev_r, jnp.where(acc_s, ids_s, 0), neg1)
                outref[s:s + 1, :] = jnp.where(greedy, tok_g, tok_r)
                numacc_g += jnp.where(prev_g, 1, 0)
                numacc_r += jnp.where(prev_r, 1, 0)
                prev_g = prev_g & match_s
                prev_r = prev_r & acc_s
            numacc_g += jnp.where(prev_g, 1, 0)
            numacc_r += jnp.where(prev_r, 1, 0)
            ok_bonus = (greedy & prev_g) | ((~greedy) & prev_r)
            outref[S:S + 1, :] = jnp.where(ok_bonus, bonus, neg1)

            numacc = jnp.where(greedy, numacc_g, numacc_r)
            nrref[...] = (S + 1) - numacc

            first_rj = numacc_r - 1
            sstar = jnp.minimum(first_rj, S - 1)
            ssref[...] = sstar
            wrref[...] = jnp.where((~greedy) & (first_rj < S), 1, 0)

            last_g = bonus
            for s in reversed(range(S)):
                last_g = jnp.where(match[s:s + 1, :], last_g,
                                   am[s:s + 1, :])
            laref[...] = jnp.where(greedy, last_g, bonus)

            msel = m[0:1, :]
            zsel = z[0:1, :]
            for s in range(1, S):
                pick = sstar == s
                msel = jnp.where(pick, m[s:s + 1, :], msel)
                zsel = jnp.where(pick, z[s:s + 1, :], zsel)
            msref[...] = msel
            zsref[...] = zsel

    small = pl.BlockSpec((S, B), lambda s, j: (0, 0))
    small1 = pl.BlockSpec((1, B), lambda s, j: (0, 0))
    return pl.pallas_call(
        body,
        grid=(S, NJ),
        in_specs=[
            pl.BlockSpec((1, C, B), lambda s, j: (s, j, 0)),
            small, small, small, small, small1, small1,
        ],
        out_specs=[
            pl.BlockSpec((S + 1, B), lambda s, j: (0, 0)),
            small1, small1, small1, small1, small1, small1,
        ],
        out_shape=[
            jax.ShapeDtypeStruct((S + 1, B), jnp.int32),
            jax.ShapeDtypeStruct((1, B), jnp.int32),
            jax.ShapeDtypeStruct((1, B), jnp.int32),
            jax.ShapeDtypeStruct((1, B), jnp.int32),
            jax.ShapeDtypeStruct((1, B), jnp.int32),
            jax.ShapeDtypeStruct((1, B), jnp.float32),
            jax.ShapeDtypeStruct((1, B), jnp.float32),
        ],
        scratch_shapes=[
            pltpu.VMEM((S, B), jnp.float32),
            pltpu.VMEM((S, B), jnp.float32),
            pltpu.VMEM((S, B), jnp.int32),
            pltpu.VMEM((1, B), jnp.float32),
            pltpu.VMEM((1, B), jnp.float32),
            pltpu.VMEM((1, B), jnp.int32),
        ],
    )(tl_t, ids_t, lat_t, dat_t, u_t, bonus_r, greedy_r)


def _k2_recover(tl_t, dp_t, q_t, sstar, wr, msel, zsel, outa, lastnw, C):
    S, V, B = tl_t.shape
    NJ = V // C

    def body(lref, dref, qref, ssref, wrref, msref, zsref, oaref, laref,
             out_ref, last_ref, gmax, gidx, gnan):
        j = pl.program_id(0)
        s = pl.program_id(1)

        @pl.when((j == 0) & (s == 0))
        def _():
            gmax[...] = jnp.full((1, B), _NEG_INF, jnp.float32)
            gidx[...] = jnp.zeros((1, B), jnp.int32)
            gnan[...] = jnp.full((1, B), V, jnp.int32)

        lanemask = (ssref[...] == s) & (wrref[...] != 0)
        X = lref[0]
        D = dref[0]
        Q = qref[...]
        p = jnp.exp(X - msref[...]) / zsref[...]
        sc = jnp.maximum(p - D, 0.0) * (1.0 / Q)
        iov = lax.broadcasted_iota(jnp.int32, (C, B), 0) + j * C
        nanm = sc != sc
        ln = jnp.min(jnp.where(nanm, iov, V), axis=0, keepdims=True)
        gnan[...] = jnp.minimum(gnan[...], jnp.where(lanemask, ln, V))
        scc = jnp.where(nanm, _NEG_INF, sc)
        lm = jnp.max(scc, axis=0, keepdims=True)
        lam = jnp.min(jnp.where(scc == lm, iov, V), axis=0, keepdims=True)
        upd = lanemask & (lm > gmax[...])
        gidx[...] = jnp.where(upd, lam, gidx[...])
        gmax[...] = jnp.where(upd, lm, gmax[...])

        @pl.when((j == NJ - 1) & (s == S - 1))
        def _():
            recn = gnan[...]
            rec = jnp.where(recn < V, recn, gidx[...])
            wrv = wrref[...] != 0
            oa = oaref[...]
            io = lax.broadcasted_iota(jnp.int32, (S + 1, B), 0)
            out_ref[...] = jnp.where((io == ssref[...]) & wrv, rec, oa)
            last_ref[...] = jnp.where(wrv, rec, laref[...])

    return pl.pallas_call(
        body,
        grid=(NJ, S),
        in_specs=[
            pl.BlockSpec((1, C, B), lambda j, s: (s, j, 0)),
            pl.BlockSpec((1, C, B), lambda j, s: (s, j, 0)),
            pl.BlockSpec((C, B), lambda j, s: (j, 0)),
            pl.BlockSpec((1, B), lambda j, s: (0, 0)),
            pl.BlockSpec((1, B), lambda j, s: (0, 0)),
            pl.BlockSpec((1, B), lambda j, s: (0, 0)),
            pl.BlockSpec((1, B), lambda j, s: (0, 0)),
            pl.BlockSpec((S + 1, B), lambda j, s: (0, 0)),
            pl.BlockSpec((1, B), lambda j, s: (0, 0)),
        ],
        out_specs=[
            pl.BlockSpec((S + 1, B), lambda j, s: (0, 0)),
            pl.BlockSpec((1, B), lambda j, s: (0, 0)),
        ],
        out_shape=[
            jax.ShapeDtypeStruct((S + 1, B), jnp.int32),
            jax.ShapeDtypeStruct((1, B), jnp.int32),
        ],
        scratch_shapes=[
            pltpu.VMEM((1, B), jnp.float32),
            pltpu.VMEM((1, B), jnp.int32),
            pltpu.VMEM((1, B), jnp.int32),
        ],
    )(tl_t, dp_t, q_t, sstar, wr, msel, zsel, outa, lastnw)


def kernel(target_logits, draft_token_ids, bonus_token_ids, is_greedy,
           uniform_probs, q, draft_probs):
    B, S = draft_token_ids.shape
    V = target_logits.shape[-1]
    idt = draft_token_ids.dtype
    C = 10000 if V % 10000 == 0 else V

    tl_t = jnp.transpose(target_logits, (1, 2, 0))
    dp_t = jnp.transpose(draft_probs, (1, 2, 0))
    q_t = jnp.transpose(q, (1, 0))

    ids_t = draft_token_ids.T.astype(jnp.int32)
    lat_f, dat_f = _sc_gather(
        ids_t.reshape(-1), tl_t.reshape(-1), dp_t.reshape(-1), V, B)

    outa, nr, lastnw, sstar, wr, msel, zsel = _k1_stats_scan(
        tl_t, ids_t, lat_f.reshape(S, B), dat_f.reshape(S, B),
        uniform_probs.T,
        bonus_token_ids.reshape(1, B).astype(jnp.int32),
        is_greedy.reshape(1, B).astype(jnp.int32), C)

    out_t, last = _k2_recover(
        tl_t, dp_t, q_t, sstar, wr, msel, zsel, outa, lastnw, C)

    return (out_t.T.astype(idt),
            nr.reshape(B).astype(jnp.int32),
            last.reshape(B).astype(idt))


# speedup vs baseline: 1.0086x; 1.0086x over previous
import functools

import jax
import jax.numpy as jnp
from jax import lax
from jax.experimental import pallas as pl
from jax.experimental.pallas import tpu as pltpu
from jax.experimental.pallas import tpu_sc as plsc

_PLACEHOLDER = -1
_NEG_INF = float("-inf")


def _sc_gather(ids_flat, lflat, dflat, V, B):
    R = ids_flat.shape[0]
    info = plsc.get_sparse_core_info()
    nw = info.num_cores * info.num_subcores
    per = R // nw
    mesh = plsc.VectorSubcoreMesh(core_axis_name="c", subcore_axis_name="s")

    @functools.partial(
        pl.kernel,
        mesh=mesh,
        out_type=(
            jax.ShapeDtypeStruct((R,), jnp.float32),
            jax.ShapeDtypeStruct((R,), jnp.float32),
        ),
        scratch_types=[
            pltpu.VMEM((per,), jnp.int32),
            pltpu.VMEM((per,), jnp.int32),
            pltpu.VMEM((per,), jnp.float32),
            pltpu.VMEM((per,), jnp.float32),
            pltpu.SemaphoreType.DMA,
            pltpu.SemaphoreType.DMA,
        ],
    )
    def k(ids_hbm, lf_hbm, df_hbm, lat_hbm, dat_hbm,
          idx_v, flat_v, lat_v, dat_v, sem1, sem2):
        wid = lax.axis_index("s") * info.num_cores + lax.axis_index("c")
        base = wid * per
        pltpu.sync_copy(ids_hbm.at[pl.ds(base, per)], idx_v)
        w_per_s = B // per
        s_w = wid // w_per_s
        b0 = (wid - s_w * w_per_s) * per
        flat_v[...] = ((s_w * V + idx_v[...]) * B + b0
                       + lax.iota(jnp.int32, per))
        cp1 = pltpu.async_copy(lf_hbm.at[flat_v], lat_v, sem1)
        cp2 = pltpu.async_copy(df_hbm.at[flat_v], dat_v, sem2)
        cp1.wait()
        cp2.wait()
        pltpu.sync_copy(lat_v, lat_hbm.at[pl.ds(base, per)])
        pltpu.sync_copy(dat_v, dat_hbm.at[pl.ds(base, per)])

    return k(ids_flat, lflat, dflat)


def _k1_stats(tl_t, C):
    S, V, B = tl_t.shape
    NJ = V // C

    def body(lref, m_out, z_out, a_out, macc, zacc, aacc):
        j = pl.program_id(1)
        X = lref[0]
        lm = jnp.max(X, axis=0, keepdims=True)
        iov = lax.broadcasted_iota(jnp.int32, (C, B), 0) + j * C
        lam = jnp.min(jnp.where(X == lm, iov, V), axis=0, keepdims=True)

        @pl.when(j == 0)
        def _():
            macc[...] = lm
            zacc[...] = jnp.sum(jnp.exp(X - lm), axis=0, keepdims=True)
            aacc[...] = lam

        @pl.when(j > 0)
        def _():
            mo = macc[...]
            mn = jnp.maximum(mo, lm)
            zacc[...] = (zacc[...] * jnp.exp(mo - mn)
                         + jnp.sum(jnp.exp(X - mn), axis=0, keepdims=True))
            aacc[...] = jnp.where(lm > mo, lam, aacc[...])
            macc[...] = mn

        @pl.when(j == NJ - 1)
        def _():
            m_out[...] = macc[...][None]
            z_out[...] = zacc[...][None]
            a_out[...] = aacc[...][None]

    return pl.pallas_call(
        body,
        grid=(S, NJ),
        in_specs=[pl.BlockSpec((1, C, B), lambda s, j: (s, j, 0))],
        out_specs=[
            pl.BlockSpec((1, 1, B), lambda s, j: (s, 0, 0)),
            pl.BlockSpec((1, 1, B), lambda s, j: (s, 0, 0)),
            pl.BlockSpec((1, 1, B), lambda s, j: (s, 0, 0)),
        ],
        out_shape=[
            jax.ShapeDtypeStruct((S, 1, B), jnp.float32),
            jax.ShapeDtypeStruct((S, 1, B), jnp.float32),
            jax.ShapeDtypeStruct((S, 1, B), jnp.int32),
        ],
        scratch_shapes=[
            pltpu.VMEM((1, B), jnp.float32),
            pltpu.VMEM((1, B), jnp.float32),
            pltpu.VMEM((1, B), jnp.int32),
        ],
    )(tl_t)


def _scan(m_t, z_t, am_t, ids_t, lat_t, dat_t, u_t, bonus_r, greedy_r):
    S, B = ids_t.shape

    def body(mref, zref, aref, iref, lref, dref, uref, bref, gref,
             outref, nrref, laref, ssref, wrref, msref, zsref):
        m = mref[...]
        z = zref[...]
        am = aref[...]
        ids = iref[...]
        lat = lref[...]
        dat = dref[...]
        u = uref[...]
        bonus = bref[...]
        greedy = gref[...] != 0

        t = jnp.exp(lat - m) / z
        acc = (dat > 0.0) & ((t / jnp.where(dat > 0.0, dat, 1.0)) >= u)
        match = ids == am

        ones = jnp.ones((1, B), dtype=jnp.bool_)
        prev_g = ones
        prev_r = ones
        numacc_g = jnp.zeros((1, B), dtype=jnp.int32)
        numacc_r = jnp.zeros((1, B), dtype=jnp.int32)
        neg1 = jnp.full((1, B), _PLACEHOLDER, dtype=jnp.int32)
        for s in range(S):
            acc_s = acc[s:s + 1, :]
            match_s = match[s:s + 1, :]
            am_s = am[s:s + 1, :]
            ids_s = ids[s:s + 1, :]
            tok_g = jnp.where(prev_g, am_s, neg1)
            tok_r = jnp.where(prev_r, jnp.where(acc_s, ids_s, 0), neg1)
            outref[s:s + 1, :] = jnp.where(greedy, tok_g, tok_r)
            numacc_g += jnp.where(prev_g, 1, 0)
            numacc_r += jnp.where(prev_r, 1, 0)
            prev_g = prev_g & match_s
            prev_r = prev_r & acc_s
        numacc_g += jnp.where(prev_g, 1, 0)
        numacc_r += jnp.where(prev_r, 1, 0)
        ok_bonus = (greedy & prev_g) | ((~greedy) & prev_r)
        outref[S:S + 1, :] = jnp.where(ok_bonus, bonus, neg1)

        numacc = jnp.where(greedy, numacc_g, numacc_r)
        nrref[...] = (S + 1) - numacc

        first_rj = numacc_r - 1
        sstar = jnp.minimum(first_rj, S - 1)
        ssref[...] = sstar
        wrref[...] = jnp.where((~greedy) & (first_rj < S), 1, 0)

        last_g = bonus
        for s in reversed(range(S)):
            last_g = jnp.where(match[s:s + 1, :], last_g, am[s:s + 1, :])
        laref[...] = jnp.where(greedy, last_g, bonus)

        msel = m[0:1, :]
        zsel = z[0:1, :]
        for s in range(1, S):
            pick = sstar == s
            msel = jnp.where(pick, m[s:s + 1, :], msel)
            zsel = jnp.where(pick, z[s:s + 1, :], zsel)
        msref[...] = msel
        zsref[...] = zsel

    return pl.pallas_call(
        body,
        out_shape=[
            jax.ShapeDtypeStruct((S + 1, B), jnp.int32),
            jax.ShapeDtypeStruct((1, B), jnp.int32),
            jax.ShapeDtypeStruct((1, B), jnp.int32),
            jax.ShapeDtypeStruct((1, B), jnp.int32),
            jax.ShapeDtypeStruct((1, B), jnp.int32),
            jax.ShapeDtypeStruct((1, B), jnp.float32),
            jax.ShapeDtypeStruct((1, B), jnp.float32),
        ],
    )(m_t, z_t, am_t, ids_t, lat_t, dat_t, u_t, bonus_r, greedy_r)


def _k2_recover(tl_t, dp_t, q_t, sstar, wr, msel, zsel, outa, lastnw, C):
    S, V, B = tl_t.shape
    NJ = V // C

    def body(lref, dref, qref, ssref, wrref, msref, zsref, oaref, laref,
             out_ref, last_ref, gmax, gidx, gnan):
        j = pl.program_id(0)
        s = pl.program_id(1)

        @pl.when((j == 0) & (s == 0))
        def _():
            gmax[...] = jnp.full((1, B), _NEG_INF, jnp.float32)
            gidx[...] = jnp.zeros((1, B), jnp.int32)
            gnan[...] = jnp.full((1, B), V, jnp.int32)

        lanemask = (ssref[...] == s) & (wrref[...] != 0)
        X = lref[0]
        D = dref[0]
        Q = qref[...]
        p = jnp.exp(X - msref[...]) / zsref[...]
        sc = jnp.maximum(p - D, 0.0) * (1.0 / Q)
        iov = lax.broadcasted_iota(jnp.int32, (C, B), 0) + j * C
        nanm = sc != sc
        ln = jnp.min(jnp.where(nanm, iov, V), axis=0, keepdims=True)
        gnan[...] = jnp.minimum(gnan[...], jnp.where(lanemask, ln, V))
        scc = jnp.where(nanm, _NEG_INF, sc)
        lm = jnp.max(scc, axis=0, keepdims=True)
        lam = jnp.min(jnp.where(scc == lm, iov, V), axis=0, keepdims=True)
        upd = lanemask & (lm > gmax[...])
        gidx[...] = jnp.where(upd, lam, gidx[...])
        gmax[...] = jnp.where(upd, lm, gmax[...])

        @pl.when((j == NJ - 1) & (s == S - 1))
        def _():
            recn = gnan[...]
            rec = jnp.where(recn < V, recn, gidx[...])
            wrv = wrref[...] != 0
            oa = oaref[...]
            io = lax.broadcasted_iota(jnp.int32, (S + 1, B), 0)
            out_ref[...] = jnp.where((io == ssref[...]) & wrv, rec, oa)
            last_ref[...] = jnp.where(wrv, rec, laref[...])

    return pl.pallas_call(
        body,
        grid=(NJ, S),
        in_specs=[
            pl.BlockSpec((1, C, B), lambda j, s: (s, j, 0)),
            pl.BlockSpec((1, C, B), lambda j, s: (s, j, 0)),
            pl.BlockSpec((C, B), lambda j, s: (j, 0)),
            pl.BlockSpec((1, B), lambda j, s: (0, 0)),
            pl.BlockSpec((1, B), lambda j, s: (0, 0)),
            pl.BlockSpec((1, B), lambda j, s: (0, 0)),
            pl.BlockSpec((1, B), lambda j, s: (0, 0)),
            pl.BlockSpec((S + 1, B), lambda j, s: (0, 0)),
            pl.BlockSpec((1, B), lambda j, s: (0, 0)),
        ],
        out_specs=[
            pl.BlockSpec((S + 1, B), lambda j, s: (0, 0)),
            pl.BlockSpec((1, B), lambda j, s: (0, 0)),
        ],
        out_shape=[
            jax.ShapeDtypeStruct((S + 1, B), jnp.int32),
            jax.ShapeDtypeStruct((1, B), jnp.int32),
        ],
        scratch_shapes=[
            pltpu.VMEM((1, B), jnp.float32),
            pltpu.VMEM((1, B), jnp.int32),
            pltpu.VMEM((1, B), jnp.int32),
        ],
    )(tl_t, dp_t, q_t, sstar, wr, msel, zsel, outa, lastnw)


def kernel(target_logits, draft_token_ids, bonus_token_ids, is_greedy,
           uniform_probs, q, draft_probs):
    B, S = draft_token_ids.shape
    V = target_logits.shape[-1]
    idt = draft_token_ids.dtype
    C = 10000 if V % 10000 == 0 else V

    tl_t = jnp.transpose(target_logits, (1, 2, 0))
    dp_t = jnp.transpose(draft_probs, (1, 2, 0))
    q_t = jnp.transpose(q, (1, 0))

    ids_t = draft_token_ids.T.astype(jnp.int32)
    lat_f, dat_f = _sc_gather(
        ids_t.reshape(-1), tl_t.reshape(-1), dp_t.reshape(-1), V, B)

    m3, z3, a3 = _k1_stats(tl_t, C)

    outa, nr, lastnw, sstar, wr, msel, zsel = _scan(
        m3.reshape(S, B), z3.reshape(S, B), a3.reshape(S, B),
        ids_t, lat_f.reshape(S, B), dat_f.reshape(S, B),
        uniform_probs.T,
        bonus_token_ids.reshape(1, B).astype(jnp.int32),
        is_greedy.reshape(1, B).astype(jnp.int32),
    )

    out_t, last = _k2_recover(
        tl_t, dp_t, q_t, sstar, wr, msel, zsel, outa, lastnw, C)

    return (out_t.T.astype(idt),
            nr.reshape(B).astype(jnp.int32),
            last.reshape(B).astype(idt))
